# baseline (speedup 1.0000x reference)
import functools

import jax
import jax.numpy as jnp
from jax import lax
from jax.experimental import pallas as pl
from jax.experimental.pallas import tpu as pltpu
from jax.experimental.pallas import tpu_sc as plsc

VOCAB = 1000000
D = 64
B = 16384
L = 50
N = B * L

_info = plsc.get_sparse_core_info()
NC = _info.num_cores
NS = _info.num_subcores
NW = NC * NS
LANES = _info.num_lanes

BBLK = 128
NBLK = B // BBLK
BLK_PER_W = NBLK // NW
GROUPS = BLK_PER_W * L
XV = BLK_PER_W * BBLK * L
SCALE = float(D) ** 0.5

_mesh = plsc.VectorSubcoreMesh(core_axis_name="c", subcore_axis_name="s")


@functools.partial(
    pl.kernel,
    out_type=jax.ShapeDtypeStruct((L * 8, NBLK, 8 * 128), jnp.float32),
    mesh=_mesh,
    scratch_types=[
        pltpu.VMEM((XV,), jnp.int32),
        pltpu.VMEM((GROUPS, BBLK), jnp.int32),
        pltpu.VMEM((BBLK, D), jnp.float32),
        pltpu.VMEM((BBLK, D), jnp.float32),
        pltpu.VMEM((8, 8 * 128), jnp.float32),
        pltpu.VMEM((8, 8 * 128), jnp.float32),
        pltpu.SemaphoreType.DMA,
        pltpu.SemaphoreType.DMA,
    ],
    compiler_params=pltpu.CompilerParams(
        use_tc_tiling_on_sc=False, needs_layout_passes=False
    ),
)
def _embed_kernel(idx_hbm, table_hbm, out_hbm, xv, idxt, rows_a, rows_b,
                  tiles_a, tiles_b, gsem, wsem):
    wid = lax.axis_index("s") * NC + lax.axis_index("c")
    iota = lax.iota(jnp.int32, LANES)
    i50 = iota * L
    row_ids = [iota + LANES * j for j in range(BBLK // LANES)]

    pltpu.sync_copy(idx_hbm.at[pl.ds(wid * XV, XV)], xv)

    def build(g, _):
        blk = g // L
        l = g % L
        for j in range(BBLK // LANES):
            ids = i50 + (blk * (BBLK * L) + j * (LANES * L) + l)
            idxt[g, pl.ds(j * LANES, LANES)] = plsc.load_gather(xv, [ids])
        return 0

    lax.fori_loop(0, GROUPS, build, 0)

    def gather_start(g, rows):
        pltpu.async_copy(table_hbm.at[idxt.at[g]], rows, gsem)

    def gather_wait(rows):
        pltpu.make_async_copy(table_hbm.at[idxt.at[0]], rows, gsem).wait()

    def write_start(g, tiles):
        blk = g // L
        l = g % L
        bhi = wid * BLK_PER_W + blk
        pltpu.async_copy(tiles, out_hbm.at[pl.ds(l * 8, 8), bhi], wsem)

    def write_wait(tiles):
        pltpu.make_async_copy(tiles, out_hbm.at[pl.ds(0, 8), 0], wsem).wait()

    def transpose(rows, tiles):
        def col(d, _):
            cid = jnp.full((LANES,), d, jnp.int32)
            dhi = d // 8
            cbase = (d % 8) * BBLK
            for j in range(BBLK // LANES):
                v = plsc.load_gather(rows, [row_ids[j], cid])
                tiles[dhi, pl.ds(cbase + j * LANES, LANES)] = v * SCALE
            return 0

        lax.fori_loop(0, D, col, 0)

    gather_start(0, rows_a)

    def step(h, _):
        g0 = 2 * h
        g1 = 2 * h + 1
        gather_wait(rows_a)
        gather_start(g1, rows_b)

        @pl.when(h >= 1)
        def _():
            write_wait(tiles_a)

        transpose(rows_a, tiles_a)
        write_start(g0, tiles_a)

        gather_wait(rows_b)

        @pl.when(g1 + 1 < GROUPS)
        def _():
            gather_start(g1 + 1, rows_a)

        @pl.when(h >= 1)
        def _():
            write_wait(tiles_b)

        transpose(rows_b, tiles_b)
        write_start(g1, tiles_b)
        return 0

    lax.fori_loop(0, GROUPS // 2, step, 0)
    write_wait(tiles_a)
    write_wait(tiles_b)


def kernel(x, embedding):
    idx = x.reshape(-1).astype(jnp.int32)
    out3 = _embed_kernel(idx, embedding)
    out5 = out3.reshape(L, 8, NBLK, 8, BBLK)
    return out5.transpose(2, 4, 0, 1, 3).reshape(B, L, D)


# speedup vs baseline: 1.7621x; 1.7621x over previous
import functools

import jax
import jax.numpy as jnp
from jax import lax
from jax.experimental import pallas as pl
from jax.experimental.pallas import tpu as pltpu
from jax.experimental.pallas import tpu_sc as plsc

VOCAB = 1000000
D = 64
B = 16384
L = 50
N = B * L

_info = plsc.get_sparse_core_info()
NC = _info.num_cores
NS = _info.num_subcores
NW = NC * NS
LANES = _info.num_lanes

BBLK = 128
NBLK = B // BBLK
BLK_PER_W = NBLK // NW
GROUPS = BLK_PER_W * L
XV = BLK_PER_W * BBLK * L
SCALE = float(D) ** 0.5

_mesh = plsc.VectorSubcoreMesh(core_axis_name="c", subcore_axis_name="s")


@functools.partial(
    pl.kernel,
    out_type=jax.ShapeDtypeStruct((L * 8, NBLK, 8 * 128), jnp.float32),
    mesh=_mesh,
    scratch_types=[
        pltpu.VMEM((XV,), jnp.int32),
        pltpu.VMEM((GROUPS, BBLK), jnp.int32),
        pltpu.VMEM((BBLK, D), jnp.float32),
        pltpu.VMEM((BBLK, D), jnp.float32),
        pltpu.VMEM((8, 8 * 128), jnp.float32),
        pltpu.VMEM((8, 8 * 128), jnp.float32),
        pltpu.SemaphoreType.DMA,
        pltpu.SemaphoreType.DMA,
    ],
    compiler_params=pltpu.CompilerParams(
        use_tc_tiling_on_sc=False, needs_layout_passes=False
    ),
)
def _embed_kernel(idx_hbm, table_hbm, out_hbm, xv, idxt, rows_a, rows_b,
                  tiles_a, tiles_b, gsem, wsem):
    wid = lax.axis_index("s") * NC + lax.axis_index("c")
    iota = lax.iota(jnp.int32, LANES)
    i50 = iota * L
    perm = [(iota + s) % LANES for s in range(LANES)]
    prow = [p // 8 for p in perm]
    pcol = [(p % 8) * BBLK + iota for p in perm]

    pltpu.sync_copy(idx_hbm.at[pl.ds(wid * XV, XV)], xv)

    def build(g, _):
        blk = g // L
        l = g % L
        for j in range(BBLK // LANES):
            ids = i50 + (blk * (BBLK * L) + j * (LANES * L) + l)
            idxt[g, pl.ds(j * LANES, LANES)] = plsc.load_gather(xv, [ids])
        return 0

    lax.fori_loop(0, GROUPS, build, 0)

    def gather_start(g, rows):
        pltpu.async_copy(table_hbm.at[idxt.at[g]], rows, gsem)

    def gather_wait(rows):
        pltpu.make_async_copy(table_hbm.at[idxt.at[0]], rows, gsem).wait()

    def write_start(g, tiles):
        blk = g // L
        l = g % L
        bhi = wid * BLK_PER_W + blk
        pltpu.async_copy(tiles, out_hbm.at[pl.ds(l * 8, 8), bhi], wsem)

    def write_wait(tiles):
        pltpu.make_async_copy(tiles, out_hbm.at[pl.ds(0, 8), 0], wsem).wait()

    def transpose(rows, tiles):
        def blk(t, _):
            jb = t // (D // LANES)
            jc = t % (D // LANES)
            rbase = iota + jb * LANES
            for s in range(LANES):
                v = plsc.load_gather(rows, [rbase, perm[s] + jc * LANES])
                plsc.store_scatter(
                    tiles,
                    [prow[s] + 2 * jc, pcol[s] + jb * LANES],
                    v * SCALE,
                )
            return 0

        lax.fori_loop(0, (BBLK // LANES) * (D // LANES), blk, 0)

    gather_start(0, rows_a)

    def step(h, _):
        g0 = 2 * h
        g1 = 2 * h + 1
        gather_wait(rows_a)
        gather_start(g1, rows_b)

        @pl.when(h >= 1)
        def _():
            write_wait(tiles_a)

        transpose(rows_a, tiles_a)
        write_start(g0, tiles_a)

        gather_wait(rows_b)

        @pl.when(g1 + 1 < GROUPS)
        def _():
            gather_start(g1 + 1, rows_a)

        @pl.when(h >= 1)
        def _():
            write_wait(tiles_b)

        transpose(rows_b, tiles_b)
        write_start(g1, tiles_b)
        return 0

    lax.fori_loop(0, GROUPS // 2, step, 0)
    write_wait(tiles_a)
    write_wait(tiles_b)


def kernel(x, embedding):
    idx = x.reshape(-1).astype(jnp.int32)
    out3 = _embed_kernel(idx, embedding)
    out5 = out3.reshape(L, 8, NBLK, 8, BBLK)
    return out5.transpose(2, 4, 0, 1, 3).reshape(B, L, D)


# speedup vs baseline: 2.7722x; 1.5732x over previous
import functools

import jax
import jax.numpy as jnp
from jax import lax
from jax.experimental import pallas as pl
from jax.experimental.pallas import tpu as pltpu
from jax.experimental.pallas import tpu_sc as plsc

VOCAB = 1000000
D = 64
B = 16384
L = 50
N = B * L

_info = plsc.get_sparse_core_info()
NC = _info.num_cores
NS = _info.num_subcores
NW = NC * NS
LANES = _info.num_lanes

BBLK = 128
NBLK = B // BBLK
BLK_PER_W = NBLK // NW
GROUPS = BLK_PER_W * L
XV = BLK_PER_W * BBLK * L
SCALE = float(D) ** 0.5

RC = 4096
GSTEPS = 123
SPLIT = RC * GSTEPS
PACK_Q = SPLIT


def _prep_body(a_ref, b_ref, o_ref):
    o_ref[:, 0:D] = jnp.swapaxes(a_ref[...], 0, 1) * SCALE
    o_ref[:, D : 2 * D] = jnp.swapaxes(b_ref[...], 0, 1) * SCALE


_prep = pl.pallas_call(
    _prep_body,
    grid=(GSTEPS,),
    in_specs=[
        pl.BlockSpec((D, RC), lambda g: (0, g)),
        pl.BlockSpec(
            (D, RC),
            lambda g: (0, jnp.minimum(GSTEPS + g, VOCAB // RC)),
        ),
    ],
    out_specs=pl.BlockSpec((RC, 2 * D), lambda g: (g, 0)),
    out_shape=jax.ShapeDtypeStruct((PACK_Q, 2 * D), jnp.float32),
)

_mesh = plsc.VectorSubcoreMesh(core_axis_name="c", subcore_axis_name="s")


@functools.partial(
    pl.kernel,
    out_type=jax.ShapeDtypeStruct((L * 8, NBLK, 8 * 128), jnp.float32),
    mesh=_mesh,
    scratch_types=[
        pltpu.VMEM((XV,), jnp.int32),
        pltpu.VMEM((GROUPS, BBLK), jnp.int32),
        pltpu.VMEM((BBLK, D), jnp.float32),
        pltpu.VMEM((BBLK, D), jnp.float32),
        pltpu.VMEM((8, 8 * 128), jnp.float32),
        pltpu.VMEM((8, 8 * 128), jnp.float32),
        pltpu.SemaphoreType.DMA,
        pltpu.SemaphoreType.DMA,
    ],
    compiler_params=pltpu.CompilerParams(
        use_tc_tiling_on_sc=False, needs_layout_passes=False
    ),
)
def _embed_kernel(idx_hbm, table_hbm, out_hbm, xv, idxt, rows_a, rows_b,
                  tiles_a, tiles_b, gsem, wsem):
    wid = lax.axis_index("s") * NC + lax.axis_index("c")
    iota = lax.iota(jnp.int32, LANES)
    i50 = iota * L
    perm = [(iota + s) % LANES for s in range(LANES)]
    prow = [p // 8 for p in perm]
    pcol = [(p % 8) * BBLK + iota for p in perm]

    pltpu.sync_copy(idx_hbm.at[pl.ds(wid * XV, XV)], xv)

    def build(g, _):
        blk = g // L
        l = g % L
        for j in range(BBLK // LANES):
            ids = i50 + (blk * (BBLK * L) + j * (LANES * L) + l)
            v = plsc.load_gather(xv, [ids])
            lr = v + v - jnp.where(v >= SPLIT, 2 * SPLIT - 1, 0)
            idxt[g, pl.ds(j * LANES, LANES)] = lr
        return 0

    lax.fori_loop(0, GROUPS, build, 0)

    def gather_start(g, rows):
        pltpu.async_copy(table_hbm.at[idxt.at[g]], rows, gsem)

    def gather_wait(rows):
        pltpu.make_async_copy(table_hbm.at[idxt.at[0]], rows, gsem).wait()

    def write_start(g, tiles):
        blk = g // L
        l = g % L
        bhi = wid * BLK_PER_W + blk
        pltpu.async_copy(tiles, out_hbm.at[pl.ds(l * 8, 8), bhi], wsem)

    def write_wait(tiles):
        pltpu.make_async_copy(tiles, out_hbm.at[pl.ds(0, 8), 0], wsem).wait()

    def transpose(rows, tiles):
        def blk(t, _):
            jb = t // (D // LANES)
            jc = t % (D // LANES)
            rbase = iota + jb * LANES
            for s in range(LANES):
                v = plsc.load_gather(rows, [rbase, perm[s] + jc * LANES])
                plsc.store_scatter(
                    tiles, [prow[s] + 2 * jc, pcol[s] + jb * LANES], v
                )
            return 0

        lax.fori_loop(0, (BBLK // LANES) * (D // LANES), blk, 0)

    gather_start(0, rows_a)

    def step(h, _):
        g0 = 2 * h
        g1 = 2 * h + 1
        gather_wait(rows_a)
        gather_start(g1, rows_b)

        @pl.when(h >= 1)
        def _():
            write_wait(tiles_a)

        transpose(rows_a, tiles_a)
        write_start(g0, tiles_a)

        gather_wait(rows_b)

        @pl.when(g1 + 1 < GROUPS)
        def _():
            gather_start(g1 + 1, rows_a)

        @pl.when(h >= 1)
        def _():
            write_wait(tiles_b)

        transpose(rows_b, tiles_b)
        write_start(g1, tiles_b)
        return 0

    lax.fori_loop(0, GROUPS // 2, step, 0)
    write_wait(tiles_a)
    write_wait(tiles_b)


def kernel(x, embedding):
    idx = x.reshape(-1).astype(jnp.int32)
    emb_t = embedding.T
    packed = _prep(emb_t, emb_t)
    table = packed.reshape(2 * PACK_Q, D)
    out3 = _embed_kernel(idx, table)
    out5 = out3.reshape(L, 8, NBLK, 8, BBLK)
    return out5.transpose(2, 4, 0, 1, 3).reshape(B, L, D)
